# Initial kernel scaffold; baseline (speedup 1.0000x reference)
import jax
import jax.numpy as jnp
from jax.experimental import pallas as pl


def kernel(x, gate_w, gate_b, w1, b1, w2, b2, w3, b3):
    raise NotImplementedError("write your pallas kernel here")


# speedup vs baseline: 2.5521x; 2.5521x over previous
import functools

import jax
import jax.numpy as jnp
from jax.experimental import pallas as pl
from jax.experimental.pallas import tpu as pltpu

T = 2048
D = 1024
F = 2048
E = 8

BT = 1024
FC = 1024
NT = T // BT
NF = F // FC


def _gelu_exact(v):
    return 0.5 * v * (1.0 + jax.lax.erf(v * 0.7071067811865476))


def _gating_body(x_ref, gw_ref, gb_ref, cw_ref):
    logits = jnp.dot(x_ref[...], gw_ref[...], preferred_element_type=jnp.float32)
    logits = logits + gb_ref[...]
    m = jnp.max(logits, axis=-1, keepdims=True)
    ex = jnp.exp(logits - m)
    soft = ex / jnp.sum(ex, axis=-1, keepdims=True)
    top = jnp.argmax(logits, axis=-1)
    iota = jax.lax.broadcasted_iota(jnp.int32, logits.shape, 1)
    oh = (iota == top[:, None]).astype(jnp.float32)
    cw_ref[...] = soft * oh


def _expert_body(x_ref, w1_ref, w2_ref, w3_ref, b1_ref, b2_ref, b3_ref,
                 cw_ref, out_ref, acc_ref):
    e = pl.program_id(1)
    f = pl.program_id(2)
    xb = x_ref[...]
    h1 = jnp.dot(xb, w1_ref[0], preferred_element_type=jnp.float32) + b1_ref[0]
    h2 = jnp.dot(xb, w2_ref[0], preferred_element_type=jnp.float32) + b2_ref[0]
    h = _gelu_exact(h1) * h2
    o = jnp.dot(h, w3_ref[0], preferred_element_type=jnp.float32)
    o = o + jnp.where(f == 0, 1.0, 0.0) * b3_ref[0]
    cwall = cw_ref[...]
    sel = (jax.lax.broadcasted_iota(jnp.int32, cwall.shape, 1) == e)
    cwcol = jnp.sum(jnp.where(sel, cwall, 0.0), axis=1, keepdims=True)
    contrib = o * cwcol
    first = (e == 0) & (f == 0)

    @pl.when(first)
    def _():
        acc_ref[...] = contrib

    @pl.when(jnp.logical_not(first))
    def _():
        acc_ref[...] = acc_ref[...] + contrib

    @pl.when((e == E - 1) & (f == NF - 1))
    def _():
        out_ref[...] = acc_ref[...]


@jax.jit
def _moe(x, gate_w, gate_b, w1, b1, w2, b2, w3, b3):
    xt = x.reshape(T, D)
    gb = gate_b.reshape(1, E)

    cw = pl.pallas_call(
        _gating_body,
        out_shape=jax.ShapeDtypeStruct((T, E), jnp.float32),
        in_specs=[
            pl.BlockSpec((T, D), lambda: (0, 0)),
            pl.BlockSpec((D, E), lambda: (0, 0)),
            pl.BlockSpec((1, E), lambda: (0, 0)),
        ],
        out_specs=pl.BlockSpec((T, E), lambda: (0, 0)),
    )(xt, gate_w, gb)

    b1r = b1.reshape(E, 1, F)
    b2r = b2.reshape(E, 1, F)
    b3r = b3.reshape(E, 1, D)

    out = pl.pallas_call(
        _expert_body,
        grid=(NT, E, NF),
        out_shape=jax.ShapeDtypeStruct((T, D), jnp.float32),
        in_specs=[
            pl.BlockSpec((BT, D), lambda t, e, f: (t, 0)),
            pl.BlockSpec((1, D, FC), lambda t, e, f: (e, 0, f)),
            pl.BlockSpec((1, D, FC), lambda t, e, f: (e, 0, f)),
            pl.BlockSpec((1, FC, D), lambda t, e, f: (e, f, 0)),
            pl.BlockSpec((1, 1, FC), lambda t, e, f: (e, 0, f)),
            pl.BlockSpec((1, 1, FC), lambda t, e, f: (e, 0, f)),
            pl.BlockSpec((1, 1, D), lambda t, e, f: (e, 0, 0)),
            pl.BlockSpec((BT, E), lambda t, e, f: (t, 0)),
        ],
        out_specs=pl.BlockSpec((BT, D), lambda t, e, f: (t, 0)),
        scratch_shapes=[pltpu.VMEM((BT, D), jnp.float32)],
        compiler_params=pltpu.CompilerParams(
            dimension_semantics=("arbitrary", "arbitrary", "arbitrary"),
        ),
    )(xt, w1, w2, w3, b1r, b2r, b3r, cw)

    final = out.reshape(1, T, D)
    aux_loss = jnp.asarray(0.0, dtype=jnp.float32)
    return (final, aux_loss)


def kernel(x, gate_w, gate_b, w1, b1, w2, b2, w3, b3):
    return _moe(x, gate_w, gate_b, w1, b1, w2, b2, w3, b3)
